# baseline (speedup 1.0000x reference)
import functools

import jax
import jax.numpy as jnp
from jax import lax
from jax.experimental import pallas as pl
from jax.experimental.pallas import tpu as pltpu
from jax.experimental.pallas import tpu_sc as plsc

B = 4096
D = 64


@functools.lru_cache(maxsize=None)
def _build_sc_gather():
    info = plsc.get_sparse_core_info()
    nc, ns = info.num_cores, info.num_subcores
    nw = nc * ns
    b_per_w = B // nw

    mesh = plsc.VectorSubcoreMesh(core_axis_name="c", subcore_axis_name="s")

    @functools.partial(
        pl.kernel,
        mesh=mesh,
        compiler_params=pltpu.CompilerParams(use_tc_tiling_on_sc=False),
        out_type=[
            jax.ShapeDtypeStruct((B, D), jnp.float32),
            jax.ShapeDtypeStruct((B, D), jnp.float32),
        ],
        scratch_types=[
            pltpu.VMEM((b_per_w,), jnp.int32),
            pltpu.VMEM((b_per_w,), jnp.int32),
            pltpu.VMEM((b_per_w, D), jnp.float32),
            pltpu.VMEM((b_per_w, D), jnp.float32),
            pltpu.SemaphoreType.DMA,
            pltpu.SemaphoreType.DMA,
        ],
    )
    def gather_kernel(uidx_hbm, iidx_hbm, utab_hbm, itab_hbm,
                      urows_out, irows_out,
                      uidx_v, iidx_v, urows_v, irows_v, sem_u, sem_i):
        wid = lax.axis_index("s") * nc + lax.axis_index("c")
        base = wid * b_per_w
        pltpu.sync_copy(uidx_hbm.at[pl.ds(base, b_per_w)], uidx_v)
        pltpu.sync_copy(iidx_hbm.at[pl.ds(base, b_per_w)], iidx_v)
        cu = pltpu.async_copy(utab_hbm.at[uidx_v], urows_v, sem_u)
        ci = pltpu.async_copy(itab_hbm.at[iidx_v], irows_v, sem_i)
        cu.wait()
        ci.wait()
        pltpu.sync_copy(urows_v, urows_out.at[pl.ds(base, b_per_w)])
        pltpu.sync_copy(irows_v, irows_out.at[pl.ds(base, b_per_w)])

    return gather_kernel


def _outer_sigmoid_body(u_ref, v_ref, o_ref):
    u = u_ref[...]
    v = v_ref[...]
    x = u[:, :, None] * v[:, None, :]
    o_ref[...] = 1.0 / (1.0 + jnp.exp(-x))


def kernel(inputs, user_table, item_table):
    u_idx = inputs[:, 0]
    i_idx = inputs[:, 1]

    u_rows, i_rows = _build_sc_gather()(u_idx, i_idx, user_table, item_table)

    bu = 256
    out = pl.pallas_call(
        _outer_sigmoid_body,
        grid=(B // bu,),
        in_specs=[
            pl.BlockSpec((bu, D), lambda i: (i, 0)),
            pl.BlockSpec((bu, D), lambda i: (i, 0)),
        ],
        out_specs=pl.BlockSpec((bu, D, D), lambda i: (i, 0, 0)),
        out_shape=jax.ShapeDtypeStruct((B, D, D), jnp.float32),
    )(u_rows, i_rows)
    return out
